# baseline (speedup 1.0000x reference)
import functools

import jax
import jax.numpy as jnp
from jax import lax
from jax.experimental import pallas as pl
from jax.experimental.pallas import tpu as pltpu
from jax.experimental.pallas import tpu_sc as plsc

_BATCH = 16384
_D = 32
_NC = 2
_NS = 16
_NW = _NC * _NS
_BPW = _BATCH // _NW
_CHUNK = 128
_NCHUNK = _BPW // _CHUNK


def _sc_gather_body(u_ids, i_ids, ut, it, u_out, i_out,
                    idx_u, idx_i, rows_u, rows_i, sem):
    wid = lax.axis_index("s") * _NC + lax.axis_index("c")
    base = wid * _NCHUNK
    pltpu.sync_copy(u_ids.at[pl.ds(base, _NCHUNK)], idx_u)
    pltpu.sync_copy(i_ids.at[pl.ds(base, _NCHUNK)], idx_i)
    copies = []
    for j in range(_NCHUNK):
        copies.append(pltpu.async_copy(ut.at[idx_u.at[j]], rows_u.at[j], sem))
        copies.append(pltpu.async_copy(it.at[idx_i.at[j]], rows_i.at[j], sem))
    for c in copies:
        c.wait()
    pltpu.sync_copy(rows_u, u_out.at[pl.ds(base, _NCHUNK)])
    pltpu.sync_copy(rows_i, i_out.at[pl.ds(base, _NCHUNK)])


_sc_gather = functools.partial(
    pl.kernel,
    mesh=plsc.VectorSubcoreMesh(core_axis_name="c", subcore_axis_name="s"),
    compiler_params=pltpu.CompilerParams(use_tc_tiling_on_sc=False),
    out_type=[
        jax.ShapeDtypeStruct((_NW * _NCHUNK, _CHUNK, _D), jnp.float32),
        jax.ShapeDtypeStruct((_NW * _NCHUNK, _CHUNK, _D), jnp.float32),
    ],
    scratch_types=[
        pltpu.VMEM((_NCHUNK, _CHUNK), jnp.int32),
        pltpu.VMEM((_NCHUNK, _CHUNK), jnp.int32),
        pltpu.VMEM((_NCHUNK, _CHUNK, _D), jnp.float32),
        pltpu.VMEM((_NCHUNK, _CHUNK, _D), jnp.float32),
        pltpu.SemaphoreType.DMA,
    ],
)(_sc_gather_body)


def _mlp_body(u_ref, i_ref, w0a, w0b, b0, w1, b1, w2, b2, w3, b3, w4, b4,
              out_ref):
    x = jnp.dot(u_ref[...], w0a[...], preferred_element_type=jnp.float32)
    x = x + jnp.dot(i_ref[...], w0b[...], preferred_element_type=jnp.float32)
    h = jnp.maximum(x + b0[...], 0.0)
    h = jnp.maximum(
        jnp.dot(h, w1[...], preferred_element_type=jnp.float32) + b1[...], 0.0)
    h = jnp.maximum(
        jnp.dot(h, w2[...], preferred_element_type=jnp.float32) + b2[...], 0.0)
    h = jnp.maximum(
        jnp.dot(h, w3[...], preferred_element_type=jnp.float32) + b3[...], 0.0)
    out_ref[...] = (
        jnp.dot(h, w4[...], preferred_element_type=jnp.float32) + b4[...])


def kernel(U_ids, I_ids, user_table, item_table,
           W0, b0, W1, b1, W2, b2, W3, b3, W4, b4):
    u_ids = U_ids.astype(jnp.int32).reshape(_NW * _NCHUNK, _CHUNK)
    i_ids = I_ids.astype(jnp.int32).reshape(_NW * _NCHUNK, _CHUNK)
    u_rows, i_rows = _sc_gather(u_ids, i_ids, user_table, item_table)
    u = u_rows.reshape(_BATCH, _D)
    i = i_rows.reshape(_BATCH, _D)
    out = pl.pallas_call(
        _mlp_body,
        out_shape=jax.ShapeDtypeStruct((_BATCH, 1), jnp.float32),
    )(u, i,
      W0[:_D], W0[_D:], b0.reshape(1, -1),
      W1, b1.reshape(1, -1),
      W2, b2.reshape(1, -1),
      W3, b3.reshape(1, -1),
      W4, b4.reshape(1, -1))
    return out


# speedup vs baseline: 1.5072x; 1.5072x over previous
import functools

import jax
import jax.numpy as jnp
from jax import lax
from jax.experimental import pallas as pl
from jax.experimental.pallas import tpu as pltpu
from jax.experimental.pallas import tpu_sc as plsc

_BATCH = 16384
_D = 32
_PK = 4
_NC = 2
_NS = 16
_NW = _NC * _NS
_BPW = _BATCH // _NW
_LPW = _BPW // _PK


def _sc_gather_body(u_ids, i_ids, ut, it, u_out, i_out,
                    sid_u, sid_i, rows, pk_u, pk_i, sem):
    wid = lax.axis_index("s") * _NC + lax.axis_index("c")
    base = wid * _BPW
    pltpu.sync_copy(u_ids.at[pl.ds(base, _BPW)], sid_u)
    pltpu.sync_copy(i_ids.at[pl.ds(base, _BPW)], sid_i)

    for tbl, sid, pk in ((ut, sid_u, pk_u), (it, sid_i, pk_i)):

        def group_body(g, _):
            v = sid[pl.ds(g * 16, 16)]
            for l in range(16):
                pltpu.async_copy(tbl.at[pl.ds(v[l], 1)],
                                 rows.at[pl.ds(g * 16 + l, 1)], sem)
            return 0

        lax.fori_loop(0, _BPW // 16, group_body, 0)

        def drain_body(r, _):
            pltpu.make_async_copy(tbl.at[pl.ds(0, 1)],
                                  rows.at[pl.ds(0, 1)], sem).wait()
            return 0

        lax.fori_loop(0, _BPW, drain_body, 0)

        def pack_body(r, _):
            ln = r // _PK
            cs = (r % _PK) * _D
            for k in range(_D // 16):
                pk[ln, pl.ds(cs + k * 16, 16)] = rows[r, pl.ds(k * 16, 16)]
            return 0

        lax.fori_loop(0, _BPW, pack_body, 0)

    pltpu.sync_copy(pk_u, u_out.at[pl.ds(wid * _LPW, _LPW)])
    pltpu.sync_copy(pk_i, i_out.at[pl.ds(wid * _LPW, _LPW)])


def _mlp_body(u_ref, i_ref, k0a, k0b, b0, k1, b1, k2, b2, k3, b3, k4, b4,
              out_ref):
    x = jnp.dot(u_ref[...], k0a[...], preferred_element_type=jnp.float32)
    x = x + jnp.dot(i_ref[...], k0b[...], preferred_element_type=jnp.float32)
    h = jnp.maximum(x + b0[...], 0.0)
    h = jnp.maximum(
        jnp.dot(h, k1[...], preferred_element_type=jnp.float32) + b1[...], 0.0)
    h = jnp.maximum(
        jnp.dot(h, k2[...], preferred_element_type=jnp.float32) + b2[...], 0.0)
    h = jnp.maximum(
        jnp.dot(h, k3[...], preferred_element_type=jnp.float32) + b3[...], 0.0)
    out_ref[...] = (
        jnp.dot(h, k4[...], preferred_element_type=jnp.float32) + b4[...])


def kernel(U_ids, I_ids, user_table, item_table,
           W0, b0, W1, b1, W2, b2, W3, b3, W4, b4):
    u_ids = U_ids.astype(jnp.int32)
    i_ids = I_ids.astype(jnp.int32)

    sc = functools.partial(
        pl.kernel,
        mesh=plsc.VectorSubcoreMesh(core_axis_name="c", subcore_axis_name="s"),
        out_type=[
            jax.ShapeDtypeStruct((_BATCH // _PK, _PK * _D), jnp.float32),
            jax.ShapeDtypeStruct((_BATCH // _PK, _PK * _D), jnp.float32),
        ],
        scratch_types=[
            pltpu.VMEM((_BPW,), jnp.int32),
            pltpu.VMEM((_BPW,), jnp.int32),
            pltpu.VMEM((_BPW, _D), jnp.float32),
            pltpu.VMEM((_LPW, _PK * _D), jnp.float32),
            pltpu.VMEM((_LPW, _PK * _D), jnp.float32),
            pltpu.SemaphoreType.DMA,
        ],
    )(_sc_gather_body)
    u_rows, i_rows = sc(u_ids, i_ids, user_table, item_table)

    eye = jnp.eye(_PK, dtype=jnp.float32)
    kr = lambda w: jnp.kron(eye, w)
    tl = lambda b: jnp.tile(b, _PK).reshape(1, -1)
    out = pl.pallas_call(
        _mlp_body,
        out_shape=jax.ShapeDtypeStruct((_BATCH // _PK, _PK), jnp.float32),
    )(u_rows, i_rows,
      kr(W0[:_D]), kr(W0[_D:]), tl(b0),
      kr(W1), tl(b1),
      kr(W2), tl(b2),
      kr(W3), tl(b3),
      kr(W4), tl(b4))
    return out.reshape(_BATCH, 1)
